# baseline (speedup 1.0000x reference)
import functools
import jax
import jax.numpy as jnp
from jax import lax
from jax.experimental import pallas as pl
from jax.experimental.pallas import tpu as pltpu
from jax.experimental.pallas import tpu_sc as plsc

_STAGE = 64


def kernel(ref_tensor, table):
    batch, _ = ref_tensor.shape
    dim = table.shape[1]
    info = plsc.get_sparse_core_info()
    nc, ns, nl = info.num_cores, info.num_subcores, info.num_lanes
    nw = nc * ns
    rows_per_w = batch // nw
    n_dmas = rows_per_w // _STAGE
    n_chunks = dim // nl
    mesh = plsc.VectorSubcoreMesh(core_axis_name="c", subcore_axis_name="s")

    @functools.partial(
        pl.kernel,
        mesh=mesh,
        out_type=jax.ShapeDtypeStruct((batch, dim), jnp.float32),
        scratch_types=[
            pltpu.VMEM((_STAGE, dim), jnp.float32),
            pltpu.SemaphoreType.DMA,
        ],
    )
    def k(table_hbm, out_hbm, stage_v, sem):
        wid = lax.axis_index("s") * nc + lax.axis_index("c")
        base = wid * rows_per_w
        pltpu.sync_copy(table_hbm, stage_v.at[0:1])

        def fill_row(r, _):
            for c in range(n_chunks):
                stage_v[r, pl.ds(c * nl, nl)] = stage_v[0, pl.ds(c * nl, nl)]
            return 0

        lax.fori_loop(1, _STAGE, fill_row, 0)

        copies = [
            pltpu.async_copy(
                stage_v, out_hbm.at[pl.ds(base + i * _STAGE, _STAGE)], sem
            )
            for i in range(n_dmas)
        ]
        for cp in copies:
            cp.wait()

    return k(table)


# speedup vs baseline: 6.4722x; 6.4722x over previous
import jax
import jax.numpy as jnp
from jax.experimental import pallas as pl
from jax.experimental.pallas import tpu as pltpu

_STAGE_ROWS = 128


def kernel(ref_tensor, table):
    batch, _ = ref_tensor.shape
    dim = table.shape[1]
    n_copies = batch // _STAGE_ROWS

    def body(table_ref, out_ref, stage, sem):
        stage[:, :] = jnp.broadcast_to(table_ref[:, :], stage.shape)
        for i in range(n_copies):
            pltpu.make_async_copy(
                stage, out_ref.at[pl.ds(i * _STAGE_ROWS, _STAGE_ROWS)], sem
            ).start()
        pltpu.make_async_copy(out_ref, out_ref, sem).wait()

    return pl.pallas_call(
        body,
        in_specs=[pl.BlockSpec(memory_space=pltpu.VMEM)],
        out_specs=pl.BlockSpec(memory_space=pltpu.MemorySpace.HBM),
        out_shape=jax.ShapeDtypeStruct((batch, dim), table.dtype),
        scratch_shapes=[
            pltpu.VMEM((_STAGE_ROWS, dim), jnp.float32),
            pltpu.SemaphoreType.DMA,
        ],
    )(table)
